# baseline (speedup 1.0000x reference)
import functools

import jax
import jax.numpy as jnp
from jax.experimental import pallas as pl

_NUM_CODES = 1024
_EMBED_DIM = 32
_N_TOKENS = 65536
_COMMITMENT_COST = 0.25
_BLOCK = 512


def _vq_body(z_ref, w_ref, zsq_ref, wsq_ref, q_ref, idx_ref, acc_ref):
    zb = z_ref[...]
    w = w_ref[...]
    mm = jax.lax.dot_general(zb, w, (((1,), (1,)), ((), ())))
    d = (zsq_ref[...] + wsq_ref[...]) - 2.0 * mm
    iota = jax.lax.broadcasted_iota(jnp.int32, (_BLOCK, _NUM_CODES), 1)
    dmin = jnp.min(d, axis=1, keepdims=True)
    idx = jnp.min(jnp.where(d == dmin, iota, _NUM_CODES), axis=1)
    idx_ref[0, 0, :] = idx
    onehot = (iota == idx[:, None]).astype(jnp.float32)
    q = jax.lax.dot_general(onehot, w, (((1,), (0,)), ((), ())),
                            precision=jax.lax.Precision.HIGHEST)
    q_ref[...] = zb + (q - zb)
    s = jnp.sum((q - zb) ** 2).reshape(1, 1)

    @pl.when(pl.program_id(0) == 0)
    def _init():
        acc_ref[...] = jnp.zeros((1, 1), jnp.float32)

    acc_ref[...] += s


@functools.partial(jax.jit, static_argnames=())
def kernel(inputs, W):
    n, d = inputs.shape
    c = W.shape[0]
    nblocks = n // _BLOCK
    inputs_sq = jnp.sum(inputs ** 2, axis=1, keepdims=True)
    embed_sq = jnp.sum(W ** 2, axis=1).reshape(1, c)

    q_st, idx3, acc = pl.pallas_call(
        _vq_body,
        grid=(nblocks,),
        in_specs=[
            pl.BlockSpec((_BLOCK, d), lambda i: (i, 0)),
            pl.BlockSpec((c, d), lambda i: (0, 0)),
            pl.BlockSpec((_BLOCK, 1), lambda i: (i, 0)),
            pl.BlockSpec((1, c), lambda i: (0, 0)),
        ],
        out_specs=[
            pl.BlockSpec((_BLOCK, d), lambda i: (i, 0)),
            pl.BlockSpec((1, 1, _BLOCK), lambda i: (i, 0, 0)),
            pl.BlockSpec((1, 1), lambda i: (0, 0)),
        ],
        out_shape=[
            jax.ShapeDtypeStruct((n, d), jnp.float32),
            jax.ShapeDtypeStruct((nblocks, 1, _BLOCK), jnp.int32),
            jax.ShapeDtypeStruct((1, 1), jnp.float32),
        ],
    )(inputs, W, inputs_sq, embed_sq)

    indices = idx3.reshape(n)
    sse = acc[0, 0]
    codebook_loss = sse / (n * d)
    commit_loss = codebook_loss
    vq_loss = codebook_loss + _COMMITMENT_COST * commit_loss
    return (q_st, indices, vq_loss, codebook_loss, commit_loss)


# speedup vs baseline: 1.3815x; 1.3815x over previous
import functools

import jax
import jax.numpy as jnp
from jax.experimental import pallas as pl

_NUM_CODES = 1024
_EMBED_DIM = 32
_N_TOKENS = 65536
_COMMITMENT_COST = 0.25
_BLOCK = 512


def _vq_body(z_ref, w_ref, zsq_ref, wsq_ref, q_ref, idx_ref, acc_ref):
    zb = z_ref[...]
    w = w_ref[...]
    mm = jax.lax.dot_general(zb, w, (((1,), (1,)), ((), ())))
    d = (zsq_ref[...] + wsq_ref[...]) - 2.0 * mm
    iota = jax.lax.broadcasted_iota(jnp.int32, (_BLOCK, _NUM_CODES), 1)
    dmin = jnp.min(d, axis=1, keepdims=True)
    idx = jnp.min(jnp.where(d == dmin, iota, _NUM_CODES), axis=1)
    idx_ref[0, 0, :] = idx
    onehot = (iota == idx[:, None]).astype(jnp.bfloat16)
    w_hi = w.astype(jnp.bfloat16)
    w_lo = (w - w_hi.astype(jnp.float32)).astype(jnp.bfloat16)
    dn = (((1,), (0,)), ((), ()))
    q = (jax.lax.dot_general(onehot, w_hi, dn,
                             preferred_element_type=jnp.float32)
         + jax.lax.dot_general(onehot, w_lo, dn,
                               preferred_element_type=jnp.float32))
    q_ref[...] = zb + (q - zb)
    s = jnp.sum((q - zb) ** 2).reshape(1, 1)

    @pl.when(pl.program_id(0) == 0)
    def _init():
        acc_ref[...] = jnp.zeros((1, 1), jnp.float32)

    acc_ref[...] += s


@functools.partial(jax.jit, static_argnames=())
def kernel(inputs, W):
    n, d = inputs.shape
    c = W.shape[0]
    nblocks = n // _BLOCK
    inputs_sq = jnp.sum(inputs ** 2, axis=1, keepdims=True)
    embed_sq = jnp.sum(W ** 2, axis=1).reshape(1, c)

    q_st, idx3, acc = pl.pallas_call(
        _vq_body,
        grid=(nblocks,),
        in_specs=[
            pl.BlockSpec((_BLOCK, d), lambda i: (i, 0)),
            pl.BlockSpec((c, d), lambda i: (0, 0)),
            pl.BlockSpec((_BLOCK, 1), lambda i: (i, 0)),
            pl.BlockSpec((1, c), lambda i: (0, 0)),
        ],
        out_specs=[
            pl.BlockSpec((_BLOCK, d), lambda i: (i, 0)),
            pl.BlockSpec((1, 1, _BLOCK), lambda i: (i, 0, 0)),
            pl.BlockSpec((1, 1), lambda i: (0, 0)),
        ],
        out_shape=[
            jax.ShapeDtypeStruct((n, d), jnp.float32),
            jax.ShapeDtypeStruct((nblocks, 1, _BLOCK), jnp.int32),
            jax.ShapeDtypeStruct((1, 1), jnp.float32),
        ],
    )(inputs, W, inputs_sq, embed_sq)

    indices = idx3.reshape(n)
    sse = acc[0, 0]
    codebook_loss = sse / (n * d)
    commit_loss = codebook_loss
    vq_loss = codebook_loss + _COMMITMENT_COST * commit_loss
    return (q_st, indices, vq_loss, codebook_loss, commit_loss)
